# baseline (speedup 1.0000x reference)
import functools

import jax
import jax.numpy as jnp
from jax import lax
from jax.experimental import pallas as pl
from jax.experimental.pallas import tpu as pltpu
from jax.experimental.pallas import tpu_sc as plsc

N = 10000
E = 320000
D = 128

NC = 2
NS = 16
NW = NC * NS
W = 56
PE = E // NW
NF = PE // W
TAIL = PE - NF * W
WD = 128
NFD = PE // WD
TAILD = PE - NFD * WD
SLAB = 640
LAST_SLAB = N - (NS - 1) * SLAB

_mesh = plsc.VectorSubcoreMesh(core_axis_name="c", subcore_axis_name="s")


@functools.partial(
    pl.kernel,
    out_type=jax.ShapeDtypeStruct((NC, N), jnp.float32),
    mesh=_mesh,
    scratch_types=[
        pltpu.VMEM((PE,), jnp.int32),
        pltpu.VMEM((WD,), jnp.int32),
        pltpu.VMEM((WD,), jnp.float32),
        pltpu.VMEM((TAILD,), jnp.int32),
        pltpu.VMEM((TAILD,), jnp.float32),
        pltpu.VMEM((SLAB,), jnp.float32),
        pltpu.VMEM_SHARED((N,), jnp.float32),
    ],
)
def _deg_sc(ei_hbm, out_hbm, dst_all, dst_v, ones_v, dst_t, ones_t, zeros_v,
            cnt_sh):
    cid = lax.axis_index("c")
    sid = lax.axis_index("s")
    wid = sid * NC + cid
    e0 = pl.multiple_of(wid * PE, 8)
    pltpu.sync_copy(ei_hbm.at[pl.ds(E + e0, PE)], dst_all)
    for i in range(WD // 16):
        ones_v[pl.ds(i * 16, 16)] = jnp.full((16,), 1.0, jnp.float32)
    ones_t[...] = jnp.full((TAILD,), 1.0, jnp.float32)

    for i in range(SLAB // 16):
        zeros_v[pl.ds(i * 16, 16)] = jnp.zeros((16,), jnp.float32)
    r0 = pl.multiple_of(sid * SLAB, 8)

    @pl.when(sid < NS - 1)
    def _():
        pltpu.sync_copy(zeros_v, cnt_sh.at[pl.ds(r0, SLAB)])

    @pl.when(sid == NS - 1)
    def _():
        pltpu.sync_copy(zeros_v.at[pl.ds(0, LAST_SLAB)],
                        cnt_sh.at[pl.ds(r0, LAST_SLAB)])

    plsc.subcore_barrier()

    def body(j, carry):
        for i in range(WD // 16):
            dst_v[pl.ds(i * 16, 16)] = dst_all[pl.ds(j * WD + i * 16, 16)]
        pltpu.sync_copy(ones_v, cnt_sh.at[dst_v], add=True)
        return carry

    lax.fori_loop(0, NFD, body, 0)
    dst_t[...] = dst_all[pl.ds(NFD * WD, TAILD)]
    pltpu.sync_copy(ones_t, cnt_sh.at[dst_t], add=True)
    plsc.subcore_barrier()

    @pl.when(sid == 0)
    def _():
        pltpu.sync_copy(cnt_sh, out_hbm.at[cid])


NB = 6


@functools.partial(
    pl.kernel,
    out_type=jax.ShapeDtypeStruct((NC, N, D), jnp.float32),
    mesh=_mesh,
    scratch_types=[
        pltpu.VMEM((NB, W), jnp.int32),
        pltpu.VMEM((NB, W), jnp.int32),
        pltpu.VMEM((NB, W, D), jnp.float32),
        pltpu.VMEM((TAIL,), jnp.int32),
        pltpu.VMEM_SHARED((N, D), jnp.float32),
        [pltpu.SemaphoreType.DMA] * NB,
        [pltpu.SemaphoreType.DMA] * NB,
        [pltpu.SemaphoreType.DMA] * NB,
    ],
)
def _agg_sc(table_hbm, ei_hbm, out_hbm, src_w, dst_w, rows_v, dst_t, acc_sh,
            sem_i, sem_g, sem_s):
    cid = lax.axis_index("c")
    sid = lax.axis_index("s")
    wid = sid * NC + cid
    r0 = pl.multiple_of(sid * SLAB, 8)
    e0 = pl.multiple_of(wid * PE, 8)

    def fire_idx(j, b):
        base = pl.multiple_of(e0 + j * W, 8)
        pltpu.async_copy(ei_hbm.at[pl.ds(base, W)], src_w.at[b], sem_i[b])
        pltpu.async_copy(ei_hbm.at[pl.ds(E + base, W)], dst_w.at[b], sem_i[b])

    def wait_idx(b):
        pltpu.make_async_copy(ei_hbm.at[pl.ds(0, W)], src_w.at[b],
                              sem_i[b]).wait()
        pltpu.make_async_copy(ei_hbm.at[pl.ds(0, W)], dst_w.at[b],
                              sem_i[b]).wait()

    def fire_gather(b):
        pltpu.async_copy(table_hbm.at[src_w.at[b]], rows_v.at[b], sem_g[b])

    def wait_gather(b):
        pltpu.make_async_copy(table_hbm.at[pl.ds(0, W)], rows_v.at[b],
                              sem_g[b]).wait()

    def fire_scatter(b):
        pltpu.async_copy(rows_v.at[b], acc_sh.at[dst_w.at[b]], sem_s[b],
                         add=True)

    def wait_scatter(b):
        pltpu.make_async_copy(rows_v.at[b], acc_sh.at[dst_w.at[b]],
                              sem_s[b]).wait()

    for k in range(5):
        fire_idx(k, k)
    for k in range(3):
        wait_idx(k)
        fire_gather(k)

    @pl.when(sid < NS - 1)
    def _():
        pltpu.sync_copy(table_hbm.at[pl.ds(r0, SLAB)], acc_sh.at[pl.ds(r0, SLAB)])

    @pl.when(sid == NS - 1)
    def _():
        pltpu.sync_copy(table_hbm.at[pl.ds(r0, LAST_SLAB)],
                        acc_sh.at[pl.ds(r0, LAST_SLAB)])

    plsc.subcore_barrier()

    def step(j, b):
        wait_gather(b)
        fire_scatter(b)

        @pl.when(j >= 1)
        def _():
            wait_scatter((b - 1) % NB)

        @pl.when(j + 5 < NF)
        def _():
            fire_idx(j + 5, (b + 5) % NB)

        @pl.when(j + 3 < NF)
        def _():
            wait_idx((b + 3) % NB)
            fire_gather((b + 3) % NB)

    def quad(g, carry):
        for b in range(NB):
            step(g * NB + b, b)
        return carry

    lax.fori_loop(0, NF // NB, quad, 0)
    for k in range(NF - (NF // NB) * NB):
        j = (NF // NB) * NB + k
        step(jnp.int32(j), j % NB)
    wait_scatter((NF - 1) % NB)

    base_t = pl.multiple_of(e0 + NF * W, 8)
    pltpu.sync_copy(ei_hbm.at[pl.ds(base_t, TAIL)], src_w.at[0, pl.ds(0, TAIL)])
    pltpu.sync_copy(ei_hbm.at[pl.ds(E + base_t, TAIL)], dst_t)
    pltpu.async_copy(table_hbm.at[src_w.at[0, pl.ds(0, TAIL)]],
                     rows_v.at[0, pl.ds(0, TAIL)], sem_g[0]).wait()
    pltpu.sync_copy(rows_v.at[0, pl.ds(0, TAIL)], acc_sh.at[dst_t], add=True)

    plsc.subcore_barrier()

    @pl.when(sid < NS - 1)
    def _():
        pltpu.sync_copy(acc_sh.at[pl.ds(r0, SLAB)], out_hbm.at[cid, pl.ds(r0, SLAB)])

    @pl.when(sid == NS - 1)
    def _():
        pltpu.sync_copy(acc_sh.at[pl.ds(r0, LAST_SLAB)],
                        out_hbm.at[cid, pl.ds(r0, LAST_SLAB)])


_RB = 1024
_GRID = (N + _RB - 1) // _RB


def _mm_body(x_ref, w_ref, h_ref):
    h_ref[:] = jnp.dot(x_ref[:], w_ref[:], preferred_element_type=jnp.float32)


def _mm_tc(x, w):
    return pl.pallas_call(
        _mm_body,
        grid=(_GRID,),
        in_specs=[
            pl.BlockSpec((_RB, D), lambda i: (i, 0)),
            pl.BlockSpec((D, D), lambda i: (0, 0)),
        ],
        out_specs=pl.BlockSpec((_RB, D), lambda i: (i, 0)),
        out_shape=jax.ShapeDtypeStruct((N, D), jnp.float32),
    )(x, w)


def _scale_body(c0_ref, c1_ref, h_ref, dis_ref, hs_ref):
    deg = c0_ref[:] + c1_ref[:] + 1.0
    dis = lax.rsqrt(deg)
    dis_ref[:] = dis
    hs_ref[:] = h_ref[:] * dis[:, None]


def _scale_tc(c0, c1, h):
    return pl.pallas_call(
        _scale_body,
        grid=(_GRID,),
        in_specs=[
            pl.BlockSpec((_RB,), lambda i: (i,)),
            pl.BlockSpec((_RB,), lambda i: (i,)),
            pl.BlockSpec((_RB, D), lambda i: (i, 0)),
        ],
        out_specs=[
            pl.BlockSpec((_RB,), lambda i: (i,)),
            pl.BlockSpec((_RB, D), lambda i: (i, 0)),
        ],
        out_shape=[
            jax.ShapeDtypeStruct((N,), jnp.float32),
            jax.ShapeDtypeStruct((N, D), jnp.float32),
        ],
    )(c0, c1, h)


def _mid_body(a0_ref, a1_ref, hs_ref, dis_ref, b_ref, w_ref, out_ref):
    tot = a0_ref[0] + a1_ref[0] - hs_ref[:]
    dis = dis_ref[:]
    o1 = jnp.maximum(tot * dis[:, None] + b_ref[:][None, :], 0.0)
    h = jnp.dot(o1, w_ref[:], preferred_element_type=jnp.float32)
    out_ref[:] = h * dis[:, None]


def _mid_tc(agg, hs, dis, b, w):
    return pl.pallas_call(
        _mid_body,
        grid=(_GRID,),
        in_specs=[
            pl.BlockSpec((1, _RB, D), lambda i: (0, i, 0)),
            pl.BlockSpec((1, _RB, D), lambda i: (1, i, 0)),
            pl.BlockSpec((_RB, D), lambda i: (i, 0)),
            pl.BlockSpec((_RB,), lambda i: (i,)),
            pl.BlockSpec((D,), lambda i: (0,)),
            pl.BlockSpec((D, D), lambda i: (0, 0)),
        ],
        out_specs=pl.BlockSpec((_RB, D), lambda i: (i, 0)),
        out_shape=jax.ShapeDtypeStruct((N, D), jnp.float32),
    )(agg, agg, hs, dis, b, w)


def _final_body(a0_ref, a1_ref, hs_ref, dis_ref, b_ref, out_ref):
    tot = a0_ref[0] + a1_ref[0] - hs_ref[:]
    out_ref[:] = tot * dis_ref[:][:, None] + b_ref[:][None, :]


def _final_tc(agg, hs, dis, b):
    return pl.pallas_call(
        _final_body,
        grid=(_GRID,),
        in_specs=[
            pl.BlockSpec((1, _RB, D), lambda i: (0, i, 0)),
            pl.BlockSpec((1, _RB, D), lambda i: (1, i, 0)),
            pl.BlockSpec((_RB, D), lambda i: (i, 0)),
            pl.BlockSpec((_RB,), lambda i: (i,)),
            pl.BlockSpec((D,), lambda i: (0,)),
        ],
        out_specs=pl.BlockSpec((_RB, D), lambda i: (i, 0)),
        out_shape=jax.ShapeDtypeStruct((N, D), jnp.float32),
    )(agg, agg, hs, dis, b)


def kernel(x, edge_index, W1, b1, W2, b2):
    ei = edge_index.astype(jnp.int32).reshape(-1)

    h1 = _mm_tc(x, W1)
    cnt = _deg_sc(ei)
    dis, h1s = _scale_tc(cnt[0], cnt[1], h1)
    agg1 = _agg_sc(h1s, ei)
    h2s = _mid_tc(agg1, h1s, dis, b1, W2)
    agg2 = _agg_sc(h2s, ei)
    return _final_tc(agg2, h2s, dis, b2)


# speedup vs baseline: 1.0330x; 1.0330x over previous
import functools

import jax
import jax.numpy as jnp
from jax import lax
from jax.experimental import pallas as pl
from jax.experimental.pallas import tpu as pltpu
from jax.experimental.pallas import tpu_sc as plsc

N = 10000
E = 320000
D = 128

NC = 2
NS = 16
NW = NC * NS
W = 96
PE = E // NW
NF = PE // W
TAIL = PE - NF * W
WD = 128
NFD = PE // WD
TAILD = PE - NFD * WD
SLAB = 640
LAST_SLAB = N - (NS - 1) * SLAB

_mesh = plsc.VectorSubcoreMesh(core_axis_name="c", subcore_axis_name="s")


@functools.partial(
    pl.kernel,
    out_type=jax.ShapeDtypeStruct((NC, N), jnp.float32),
    mesh=_mesh,
    scratch_types=[
        pltpu.VMEM((PE,), jnp.int32),
        pltpu.VMEM((WD,), jnp.int32),
        pltpu.VMEM((WD,), jnp.float32),
        pltpu.VMEM((TAILD,), jnp.int32),
        pltpu.VMEM((TAILD,), jnp.float32),
        pltpu.VMEM((SLAB,), jnp.float32),
        pltpu.VMEM_SHARED((N,), jnp.float32),
    ],
)
def _deg_sc(ei_hbm, out_hbm, dst_all, dst_v, ones_v, dst_t, ones_t, zeros_v,
            cnt_sh):
    cid = lax.axis_index("c")
    sid = lax.axis_index("s")
    wid = sid * NC + cid
    e0 = pl.multiple_of(wid * PE, 8)
    pltpu.sync_copy(ei_hbm.at[pl.ds(E + e0, PE)], dst_all)
    for i in range(WD // 16):
        ones_v[pl.ds(i * 16, 16)] = jnp.full((16,), 1.0, jnp.float32)
    ones_t[...] = jnp.full((TAILD,), 1.0, jnp.float32)

    for i in range(SLAB // 16):
        zeros_v[pl.ds(i * 16, 16)] = jnp.zeros((16,), jnp.float32)
    r0 = pl.multiple_of(sid * SLAB, 8)

    @pl.when(sid < NS - 1)
    def _():
        pltpu.sync_copy(zeros_v, cnt_sh.at[pl.ds(r0, SLAB)])

    @pl.when(sid == NS - 1)
    def _():
        pltpu.sync_copy(zeros_v.at[pl.ds(0, LAST_SLAB)],
                        cnt_sh.at[pl.ds(r0, LAST_SLAB)])

    plsc.subcore_barrier()

    def body(j, carry):
        for i in range(WD // 16):
            dst_v[pl.ds(i * 16, 16)] = dst_all[pl.ds(j * WD + i * 16, 16)]
        pltpu.sync_copy(ones_v, cnt_sh.at[dst_v], add=True)
        return carry

    lax.fori_loop(0, NFD, body, 0)
    dst_t[...] = dst_all[pl.ds(NFD * WD, TAILD)]
    pltpu.sync_copy(ones_t, cnt_sh.at[dst_t], add=True)
    plsc.subcore_barrier()

    @pl.when(sid == 0)
    def _():
        pltpu.sync_copy(cnt_sh, out_hbm.at[cid])


NB = 4


@functools.partial(
    pl.kernel,
    out_type=jax.ShapeDtypeStruct((NC, N, D), jnp.float32),
    mesh=_mesh,
    scratch_types=[
        pltpu.VMEM((NB, W), jnp.int32),
        pltpu.VMEM((NB, W), jnp.int32),
        pltpu.VMEM((NB, W, D), jnp.float32),
        pltpu.VMEM((TAIL,), jnp.int32),
        pltpu.VMEM_SHARED((N, D), jnp.float32),
        [pltpu.SemaphoreType.DMA] * NB,
        [pltpu.SemaphoreType.DMA] * NB,
        [pltpu.SemaphoreType.DMA] * NB,
    ],
)
def _agg_sc(table_hbm, ei_hbm, out_hbm, src_w, dst_w, rows_v, dst_t, acc_sh,
            sem_i, sem_g, sem_s):
    cid = lax.axis_index("c")
    sid = lax.axis_index("s")
    wid = sid * NC + cid
    r0 = pl.multiple_of(sid * SLAB, 8)
    e0 = pl.multiple_of(wid * PE, 8)

    def fire_idx(j, b):
        base = pl.multiple_of(e0 + j * W, 8)
        pltpu.async_copy(ei_hbm.at[pl.ds(base, W)], src_w.at[b], sem_i[b])
        pltpu.async_copy(ei_hbm.at[pl.ds(E + base, W)], dst_w.at[b], sem_i[b])

    def wait_idx(b):
        pltpu.make_async_copy(ei_hbm.at[pl.ds(0, W)], src_w.at[b],
                              sem_i[b]).wait()
        pltpu.make_async_copy(ei_hbm.at[pl.ds(0, W)], dst_w.at[b],
                              sem_i[b]).wait()

    def fire_gather(b):
        pltpu.async_copy(table_hbm.at[src_w.at[b]], rows_v.at[b], sem_g[b])

    def wait_gather(b):
        pltpu.make_async_copy(table_hbm.at[pl.ds(0, W)], rows_v.at[b],
                              sem_g[b]).wait()

    def fire_scatter(b):
        pltpu.async_copy(rows_v.at[b], acc_sh.at[dst_w.at[b]], sem_s[b],
                         add=True)

    def wait_scatter(b):
        pltpu.make_async_copy(rows_v.at[b], acc_sh.at[dst_w.at[b]],
                              sem_s[b]).wait()

    for k in range(3):
        fire_idx(k, k)
    for k in range(2):
        wait_idx(k)
        fire_gather(k)

    @pl.when(sid < NS - 1)
    def _():
        pltpu.sync_copy(table_hbm.at[pl.ds(r0, SLAB)], acc_sh.at[pl.ds(r0, SLAB)])

    @pl.when(sid == NS - 1)
    def _():
        pltpu.sync_copy(table_hbm.at[pl.ds(r0, LAST_SLAB)],
                        acc_sh.at[pl.ds(r0, LAST_SLAB)])

    plsc.subcore_barrier()

    def step(j, b):
        wait_gather(b)
        fire_scatter(b)

        @pl.when(j >= 1)
        def _():
            wait_scatter((b - 1) % NB)

        @pl.when(j + 3 < NF)
        def _():
            fire_idx(j + 3, (b + 3) % NB)

        @pl.when(j + 2 < NF)
        def _():
            wait_idx((b + 2) % NB)
            fire_gather((b + 2) % NB)

    def quad(g, carry):
        for b in range(NB):
            step(g * NB + b, b)
        return carry

    lax.fori_loop(0, NF // NB, quad, 0)
    for k in range(NF - (NF // NB) * NB):
        j = (NF // NB) * NB + k
        step(jnp.int32(j), j % NB)
    wait_scatter((NF - 1) % NB)

    base_t = pl.multiple_of(e0 + NF * W, 8)
    pltpu.sync_copy(ei_hbm.at[pl.ds(base_t, TAIL)], src_w.at[0, pl.ds(0, TAIL)])
    pltpu.sync_copy(ei_hbm.at[pl.ds(E + base_t, TAIL)], dst_t)
    pltpu.async_copy(table_hbm.at[src_w.at[0, pl.ds(0, TAIL)]],
                     rows_v.at[0, pl.ds(0, TAIL)], sem_g[0]).wait()
    pltpu.sync_copy(rows_v.at[0, pl.ds(0, TAIL)], acc_sh.at[dst_t], add=True)

    plsc.subcore_barrier()

    @pl.when(sid < NS - 1)
    def _():
        pltpu.sync_copy(acc_sh.at[pl.ds(r0, SLAB)], out_hbm.at[cid, pl.ds(r0, SLAB)])

    @pl.when(sid == NS - 1)
    def _():
        pltpu.sync_copy(acc_sh.at[pl.ds(r0, LAST_SLAB)],
                        out_hbm.at[cid, pl.ds(r0, LAST_SLAB)])


_RB = 1024
_GRID = (N + _RB - 1) // _RB


def _mm_body(x_ref, w_ref, h_ref):
    h_ref[:] = jnp.dot(x_ref[:], w_ref[:], preferred_element_type=jnp.float32)


def _mm_tc(x, w):
    return pl.pallas_call(
        _mm_body,
        grid=(_GRID,),
        in_specs=[
            pl.BlockSpec((_RB, D), lambda i: (i, 0)),
            pl.BlockSpec((D, D), lambda i: (0, 0)),
        ],
        out_specs=pl.BlockSpec((_RB, D), lambda i: (i, 0)),
        out_shape=jax.ShapeDtypeStruct((N, D), jnp.float32),
    )(x, w)


def _scale_body(c0_ref, c1_ref, h_ref, dis_ref, hs_ref):
    deg = c0_ref[:] + c1_ref[:] + 1.0
    dis = lax.rsqrt(deg)
    dis_ref[:] = dis
    hs_ref[:] = h_ref[:] * dis[:, None]


def _scale_tc(c0, c1, h):
    return pl.pallas_call(
        _scale_body,
        grid=(_GRID,),
        in_specs=[
            pl.BlockSpec((_RB,), lambda i: (i,)),
            pl.BlockSpec((_RB,), lambda i: (i,)),
            pl.BlockSpec((_RB, D), lambda i: (i, 0)),
        ],
        out_specs=[
            pl.BlockSpec((_RB,), lambda i: (i,)),
            pl.BlockSpec((_RB, D), lambda i: (i, 0)),
        ],
        out_shape=[
            jax.ShapeDtypeStruct((N,), jnp.float32),
            jax.ShapeDtypeStruct((N, D), jnp.float32),
        ],
    )(c0, c1, h)


def _mid_body(a0_ref, a1_ref, hs_ref, dis_ref, b_ref, w_ref, out_ref):
    tot = a0_ref[0] + a1_ref[0] - hs_ref[:]
    dis = dis_ref[:]
    o1 = jnp.maximum(tot * dis[:, None] + b_ref[:][None, :], 0.0)
    h = jnp.dot(o1, w_ref[:], preferred_element_type=jnp.float32)
    out_ref[:] = h * dis[:, None]


def _mid_tc(agg, hs, dis, b, w):
    return pl.pallas_call(
        _mid_body,
        grid=(_GRID,),
        in_specs=[
            pl.BlockSpec((1, _RB, D), lambda i: (0, i, 0)),
            pl.BlockSpec((1, _RB, D), lambda i: (1, i, 0)),
            pl.BlockSpec((_RB, D), lambda i: (i, 0)),
            pl.BlockSpec((_RB,), lambda i: (i,)),
            pl.BlockSpec((D,), lambda i: (0,)),
            pl.BlockSpec((D, D), lambda i: (0, 0)),
        ],
        out_specs=pl.BlockSpec((_RB, D), lambda i: (i, 0)),
        out_shape=jax.ShapeDtypeStruct((N, D), jnp.float32),
    )(agg, agg, hs, dis, b, w)


def _final_body(a0_ref, a1_ref, hs_ref, dis_ref, b_ref, out_ref):
    tot = a0_ref[0] + a1_ref[0] - hs_ref[:]
    out_ref[:] = tot * dis_ref[:][:, None] + b_ref[:][None, :]


def _final_tc(agg, hs, dis, b):
    return pl.pallas_call(
        _final_body,
        grid=(_GRID,),
        in_specs=[
            pl.BlockSpec((1, _RB, D), lambda i: (0, i, 0)),
            pl.BlockSpec((1, _RB, D), lambda i: (1, i, 0)),
            pl.BlockSpec((_RB, D), lambda i: (i, 0)),
            pl.BlockSpec((_RB,), lambda i: (i,)),
            pl.BlockSpec((D,), lambda i: (0,)),
        ],
        out_specs=pl.BlockSpec((_RB, D), lambda i: (i, 0)),
        out_shape=jax.ShapeDtypeStruct((N, D), jnp.float32),
    )(agg, agg, hs, dis, b)


def kernel(x, edge_index, W1, b1, W2, b2):
    ei = edge_index.astype(jnp.int32).reshape(-1)

    h1 = _mm_tc(x, W1)
    cnt = _deg_sc(ei)
    dis, h1s = _scale_tc(cnt[0], cnt[1], h1)
    agg1 = _agg_sc(h1s, ei)
    h2s = _mid_tc(agg1, h1s, dis, b1, W2)
    agg2 = _agg_sc(h2s, ei)
    return _final_tc(agg2, h2s, dis, b2)


# speedup vs baseline: 1.0587x; 1.0249x over previous
import functools

import jax
import jax.numpy as jnp
from jax import lax
from jax.experimental import pallas as pl
from jax.experimental.pallas import tpu as pltpu
from jax.experimental.pallas import tpu_sc as plsc

N = 10000
E = 320000
D = 128

NC = 2
NS = 16
NW = NC * NS
W = 128
PE = 9984
NF = PE // W
XW = E - NW * PE
XWIN = XW // W
SLAB = 640
LAST_SLAB = N - (NS - 1) * SLAB

_mesh = plsc.VectorSubcoreMesh(core_axis_name="c", subcore_axis_name="s")


@functools.partial(
    pl.kernel,
    out_type=jax.ShapeDtypeStruct((NC, N), jnp.float32),
    mesh=_mesh,
    scratch_types=[
        pltpu.VMEM((2, PE), jnp.int32),
        pltpu.VMEM((W,), jnp.int32),
        pltpu.VMEM((W,), jnp.float32),
        pltpu.VMEM((2, W), jnp.int32),
        pltpu.VMEM((SLAB,), jnp.float32),
        pltpu.VMEM_SHARED((N,), jnp.float32),
    ],
)
def _deg_sc(ei_hbm, out_hbm, ei_all, dst_v, ones_v, idx_x, zeros_v, cnt_sh):
    cid = lax.axis_index("c")
    sid = lax.axis_index("s")
    wid = sid * NC + cid
    e0 = pl.multiple_of(wid * PE, W)
    pltpu.sync_copy(ei_hbm.at[pl.ds(0, 2), pl.ds(e0, PE)], ei_all)
    for i in range(W // 16):
        ones_v[pl.ds(i * 16, 16)] = jnp.full((16,), 1.0, jnp.float32)

    for i in range(SLAB // 16):
        zeros_v[pl.ds(i * 16, 16)] = jnp.zeros((16,), jnp.float32)
    r0 = pl.multiple_of(sid * SLAB, 8)

    @pl.when(sid < NS - 1)
    def _():
        pltpu.sync_copy(zeros_v, cnt_sh.at[pl.ds(r0, SLAB)])

    @pl.when(sid == NS - 1)
    def _():
        pltpu.sync_copy(zeros_v.at[pl.ds(0, LAST_SLAB)],
                        cnt_sh.at[pl.ds(r0, LAST_SLAB)])

    plsc.subcore_barrier()

    def body(j, carry):
        for i in range(W // 16):
            dst_v[pl.ds(i * 16, 16)] = ei_all[1, pl.ds(j * W + i * 16, 16)]
        pltpu.sync_copy(ones_v, cnt_sh.at[dst_v], add=True)
        return carry

    lax.fori_loop(0, NF, body, 0)

    @pl.when(wid < XWIN)
    def _():
        basex = pl.multiple_of(NW * PE + wid * W, W)
        pltpu.sync_copy(ei_hbm.at[pl.ds(0, 2), pl.ds(basex, W)], idx_x)
        pltpu.sync_copy(ones_v, cnt_sh.at[idx_x.at[1]], add=True)

    plsc.subcore_barrier()

    @pl.when(sid == 0)
    def _():
        pltpu.sync_copy(cnt_sh, out_hbm.at[cid])


NR = 3
NI = 4


@functools.partial(
    pl.kernel,
    out_type=jax.ShapeDtypeStruct((NC, N, D), jnp.float32),
    mesh=_mesh,
    scratch_types=[
        pltpu.VMEM((NI, 2, W), jnp.int32),
        pltpu.VMEM((NR, W, D), jnp.float32),
        pltpu.VMEM_SHARED((N, D), jnp.float32),
        [pltpu.SemaphoreType.DMA] * NI,
        [pltpu.SemaphoreType.DMA] * NR,
        [pltpu.SemaphoreType.DMA] * NR,
    ],
)
def _agg_sc(table_hbm, ei_hbm, out_hbm, idx_w, rows_v, acc_sh,
            sem_i, sem_g, sem_s):
    cid = lax.axis_index("c")
    sid = lax.axis_index("s")
    wid = sid * NC + cid
    r0 = pl.multiple_of(sid * SLAB, 8)
    e0 = pl.multiple_of(wid * PE, W)

    def fire_idx(j, ib):
        base = pl.multiple_of(e0 + j * W, W)
        pltpu.async_copy(ei_hbm.at[pl.ds(0, 2), pl.ds(base, W)], idx_w.at[ib],
                         sem_i[ib])

    def wait_idx(ib):
        pltpu.make_async_copy(ei_hbm.at[pl.ds(0, 2), pl.ds(0, W)],
                              idx_w.at[ib], sem_i[ib]).wait()

    def fire_gather(rb, ib):
        pltpu.async_copy(table_hbm.at[idx_w.at[ib, 0]], rows_v.at[rb],
                         sem_g[rb])

    def wait_gather(rb):
        pltpu.make_async_copy(table_hbm.at[pl.ds(0, W)], rows_v.at[rb],
                              sem_g[rb]).wait()

    def fire_scatter(rb, ib):
        pltpu.async_copy(rows_v.at[rb], acc_sh.at[idx_w.at[ib, 1]], sem_s[rb],
                         add=True)

    def wait_scatter(rb, ib):
        pltpu.make_async_copy(rows_v.at[rb], acc_sh.at[idx_w.at[ib, 1]],
                              sem_s[rb]).wait()

    for k in range(3):
        fire_idx(k, k)
    for k in range(2):
        wait_idx(k)
        fire_gather(k, k)

    @pl.when(sid < NS - 1)
    def _():
        pltpu.sync_copy(table_hbm.at[pl.ds(r0, SLAB)], acc_sh.at[pl.ds(r0, SLAB)])

    @pl.when(sid == NS - 1)
    def _():
        pltpu.sync_copy(table_hbm.at[pl.ds(r0, LAST_SLAB)],
                        acc_sh.at[pl.ds(r0, LAST_SLAB)])

    plsc.subcore_barrier()

    def step(j, rb, ib):
        wait_gather(rb)
        fire_scatter(rb, ib)

        @pl.when(j >= 1)
        def _():
            wait_scatter((rb - 1) % NR, (ib - 1) % NI)

        @pl.when(j + 3 < NF)
        def _():
            fire_idx(j + 3, (ib + 3) % NI)

        @pl.when(j + 2 < NF)
        def _():
            wait_idx((ib + 2) % NI)
            fire_gather((rb + 2) % NR, (ib + 2) % NI)

    def dozen(g, carry):
        for k in range(12):
            step(g * 12 + k, k % NR, k % NI)
        return carry

    lax.fori_loop(0, NF // 12, dozen, 0)
    for k in range(NF - (NF // 12) * 12):
        j = (NF // 12) * 12 + k
        step(jnp.int32(j), j % NR, j % NI)
    wait_scatter((NF - 1) % NR, (NF - 1) % NI)

    @pl.when(wid < XWIN)
    def _():
        basex = pl.multiple_of(NW * PE + wid * W, W)
        pltpu.sync_copy(ei_hbm.at[pl.ds(0, 2), pl.ds(basex, W)], idx_w.at[0])
        pltpu.async_copy(table_hbm.at[idx_w.at[0, 0]], rows_v.at[0],
                         sem_g[0]).wait()
        pltpu.sync_copy(rows_v.at[0], acc_sh.at[idx_w.at[0, 1]], add=True)

    plsc.subcore_barrier()

    @pl.when(sid < NS - 1)
    def _():
        pltpu.sync_copy(acc_sh.at[pl.ds(r0, SLAB)], out_hbm.at[cid, pl.ds(r0, SLAB)])

    @pl.when(sid == NS - 1)
    def _():
        pltpu.sync_copy(acc_sh.at[pl.ds(r0, LAST_SLAB)],
                        out_hbm.at[cid, pl.ds(r0, LAST_SLAB)])


_RB = 2048
_GRID = (N + _RB - 1) // _RB


def _mm_body(x_ref, w_ref, h_ref):
    h_ref[:] = jnp.dot(x_ref[:], w_ref[:], preferred_element_type=jnp.float32)


def _mm_tc(x, w):
    return pl.pallas_call(
        _mm_body,
        grid=(_GRID,),
        in_specs=[
            pl.BlockSpec((_RB, D), lambda i: (i, 0)),
            pl.BlockSpec((D, D), lambda i: (0, 0)),
        ],
        out_specs=pl.BlockSpec((_RB, D), lambda i: (i, 0)),
        out_shape=jax.ShapeDtypeStruct((N, D), jnp.float32),
    )(x, w)


def _scale_body(c0_ref, c1_ref, h_ref, dis_ref, hs_ref):
    deg = c0_ref[:] + c1_ref[:] + 1.0
    dis = lax.rsqrt(deg)
    dis_ref[:] = dis
    hs_ref[:] = h_ref[:] * dis[:, None]


def _scale_tc(c0, c1, h):
    return pl.pallas_call(
        _scale_body,
        grid=(_GRID,),
        in_specs=[
            pl.BlockSpec((_RB,), lambda i: (i,)),
            pl.BlockSpec((_RB,), lambda i: (i,)),
            pl.BlockSpec((_RB, D), lambda i: (i, 0)),
        ],
        out_specs=[
            pl.BlockSpec((_RB,), lambda i: (i,)),
            pl.BlockSpec((_RB, D), lambda i: (i, 0)),
        ],
        out_shape=[
            jax.ShapeDtypeStruct((N,), jnp.float32),
            jax.ShapeDtypeStruct((N, D), jnp.float32),
        ],
    )(c0, c1, h)


def _mid_body(a0_ref, a1_ref, hs_ref, dis_ref, b_ref, w_ref, out_ref):
    tot = a0_ref[0] + a1_ref[0] - hs_ref[:]
    dis = dis_ref[:]
    o1 = jnp.maximum(tot * dis[:, None] + b_ref[:][None, :], 0.0)
    h = jnp.dot(o1, w_ref[:], preferred_element_type=jnp.float32)
    out_ref[:] = h * dis[:, None]


def _mid_tc(agg, hs, dis, b, w):
    return pl.pallas_call(
        _mid_body,
        grid=(_GRID,),
        in_specs=[
            pl.BlockSpec((1, _RB, D), lambda i: (0, i, 0)),
            pl.BlockSpec((1, _RB, D), lambda i: (1, i, 0)),
            pl.BlockSpec((_RB, D), lambda i: (i, 0)),
            pl.BlockSpec((_RB,), lambda i: (i,)),
            pl.BlockSpec((D,), lambda i: (0,)),
            pl.BlockSpec((D, D), lambda i: (0, 0)),
        ],
        out_specs=pl.BlockSpec((_RB, D), lambda i: (i, 0)),
        out_shape=jax.ShapeDtypeStruct((N, D), jnp.float32),
    )(agg, agg, hs, dis, b, w)


def _final_body(a0_ref, a1_ref, hs_ref, dis_ref, b_ref, out_ref):
    tot = a0_ref[0] + a1_ref[0] - hs_ref[:]
    out_ref[:] = tot * dis_ref[:][:, None] + b_ref[:][None, :]


def _final_tc(agg, hs, dis, b):
    return pl.pallas_call(
        _final_body,
        grid=(_GRID,),
        in_specs=[
            pl.BlockSpec((1, _RB, D), lambda i: (0, i, 0)),
            pl.BlockSpec((1, _RB, D), lambda i: (1, i, 0)),
            pl.BlockSpec((_RB, D), lambda i: (i, 0)),
            pl.BlockSpec((_RB,), lambda i: (i,)),
            pl.BlockSpec((D,), lambda i: (0,)),
        ],
        out_specs=pl.BlockSpec((_RB, D), lambda i: (i, 0)),
        out_shape=jax.ShapeDtypeStruct((N, D), jnp.float32),
    )(agg, agg, hs, dis, b)


def kernel(x, edge_index, W1, b1, W2, b2):
    ei = edge_index.astype(jnp.int32)

    h1 = _mm_tc(x, W1)
    cnt = _deg_sc(ei)
    dis, h1s = _scale_tc(cnt[0], cnt[1], h1)
    agg1 = _agg_sc(h1s, ei)
    h2s = _mid_tc(agg1, h1s, dis, b1, W2)
    agg2 = _agg_sc(h2s, ei)
    return _final_tc(agg2, h2s, dis, b2)
